# Initial kernel scaffold; baseline (speedup 1.0000x reference)
import jax
import jax.numpy as jnp
from jax.experimental import pallas as pl


def kernel(x, adj, W1, b1, W2, b2):
    raise NotImplementedError("write your pallas kernel here")


# speedup vs baseline: 11030.9038x; 11030.9038x over previous
import jax
import jax.numpy as jnp
from jax.experimental import pallas as pl
from jax.experimental.pallas import tpu as pltpu


def _cog_kernel(xt_ref, adj_ref, w1_ref, b1_ref, w2_ref, b2_ref, out_ref):
    adj = adj_ref[...]
    deg = jnp.sum(adj, axis=0, keepdims=True) + 1.0
    dinv = jax.lax.rsqrt(deg)

    xw1 = jnp.dot(w1_ref[...], xt_ref[...], preferred_element_type=jnp.float32)
    v1 = xw1 * dinv
    agg1 = jnp.dot(v1, adj, preferred_element_type=jnp.float32) + v1
    h1 = jnp.maximum(agg1 * dinv + b1_ref[...], 0.0)

    xw2 = jnp.dot(w2_ref[...], h1, preferred_element_type=jnp.float32)
    v2 = xw2 * dinv
    agg2 = jnp.dot(v2, adj, preferred_element_type=jnp.float32) + v2
    z = (agg2 * dinv + b2_ref[...]) * 5.0

    m = jnp.max(z, axis=0, keepdims=True)
    lse = jnp.log(jnp.sum(jnp.exp(z - m), axis=0, keepdims=True)) + m
    out_ref[...] = z - lse


def kernel(x, adj, W1, b1, W2, b2):
    n, _ = x.shape
    nclass = W2.shape[0]
    out_t = pl.pallas_call(
        _cog_kernel,
        out_shape=jax.ShapeDtypeStruct((nclass, n), jnp.float32),
    )(x.T, adj, W1, b1.reshape(-1, 1), W2, b2.reshape(-1, 1))
    return out_t.T
